# Initial kernel scaffold; baseline (speedup 1.0000x reference)
import jax
import jax.numpy as jnp
from jax.experimental import pallas as pl


def kernel(x, pos, edge_index, params):
    raise NotImplementedError("write your pallas kernel here")


# speedup vs baseline: 2.7939x; 2.7939x over previous
import functools

import jax
import jax.numpy as jnp
from jax import lax
from jax.experimental import pallas as pl
from jax.experimental.pallas import tpu as pltpu
from jax.experimental.pallas import tpu_sc as plsc

N = 10000
E = 320000
D = 128
P = 8

NC = 2
NS = 16
NW = NC * NS
EPW = E // NW
CH = 80
NCHUNK = EPW // CH
ZR = 400
NZC = N // ZR

@functools.cache
def _sc_mesh():
  return plsc.VectorSubcoreMesh(
      core_axis_name="c", subcore_axis_name="s", num_cores=NC, num_subcores=NS)


_f32 = jnp.float32


def _sc_gather(row, col, a_t, b_t, posp):
  @functools.partial(
      pl.kernel,
      out_type=[
          jax.ShapeDtypeStruct((E, D), _f32),
          jax.ShapeDtypeStruct((E, D), _f32),
          jax.ShapeDtypeStruct((E, D), _f32),
          jax.ShapeDtypeStruct((E, D), _f32),
      ],
      mesh=_sc_mesh(),
      scratch_types=[
          pltpu.VMEM((CH,), jnp.int32),
          pltpu.VMEM((CH,), jnp.int32),
          pltpu.VMEM((CH, D), _f32),
          pltpu.VMEM((CH, D), _f32),
          pltpu.VMEM((CH, D), _f32),
          pltpu.VMEM((CH, D), _f32),
          pltpu.SemaphoreType.DMA,
          pltpu.SemaphoreType.DMA,
          pltpu.SemaphoreType.DMA,
          pltpu.SemaphoreType.DMA,
      ],
  )
  def k(row_h, col_h, a_h, b_h, p_h, ga_h, gb_h, pr_h, pc_h,
        idx_r, idx_c, buf_a, buf_b, buf_r, buf_c, sa, sb, sr, sc_):
    wid = lax.axis_index("s") * NC + lax.axis_index("c")

    def body(i, _):
      base = wid * EPW + i * CH
      pltpu.sync_copy(row_h.at[pl.ds(base, CH)], idx_r)
      pltpu.sync_copy(col_h.at[pl.ds(base, CH)], idx_c)
      ca = pltpu.async_copy(a_h.at[idx_r], buf_a, sa)
      cb = pltpu.async_copy(b_h.at[idx_c], buf_b, sb)
      cr = pltpu.async_copy(p_h.at[idx_r], buf_r, sr)
      cc = pltpu.async_copy(p_h.at[idx_c], buf_c, sc_)
      ca.wait()
      cb.wait()
      cr.wait()
      cc.wait()
      pltpu.sync_copy(buf_a, ga_h.at[pl.ds(base, CH)])
      pltpu.sync_copy(buf_b, gb_h.at[pl.ds(base, CH)])
      pltpu.sync_copy(buf_r, pr_h.at[pl.ds(base, CH)])
      pltpu.sync_copy(buf_c, pc_h.at[pl.ds(base, CH)])
      return 0

    lax.fori_loop(0, NCHUNK, body, 0)

  return k(row, col, a_t, b_t, posp)


def _sc_scatter(row, z_m, *vals):
  nv = len(vals)

  @functools.partial(
      pl.kernel,
      out_type=[jax.ShapeDtypeStruct((NC, N, D), _f32)] * nv,
      mesh=_sc_mesh(),
      scratch_types=[
          pltpu.VMEM((CH,), jnp.int32),
          pltpu.VMEM((CH, D), _f32),
          pltpu.VMEM_SHARED((N, D), _f32),
      ],
  )
  def k(row_h, zm_h, *rest):
    val_hs = rest[:nv]
    out_hs = rest[nv:2 * nv]
    idx, buf_m, acc_m = rest[2 * nv:]
    cid = lax.axis_index("c")
    sid = lax.axis_index("s")
    wid = sid * NC + cid

    for v in range(nv):
      def zbody(j, _):
        c = sid + NS * j

        @pl.when(c < NZC)
        def _():
          pltpu.sync_copy(zm_h, acc_m.at[pl.ds(c * ZR, ZR)])
        return 0

      lax.fori_loop(0, (NZC + NS - 1) // NS, zbody, 0)
      plsc.subcore_barrier()

      def sbody(i, _, v=v):
        base = wid * EPW + i * CH
        pltpu.sync_copy(row_h.at[pl.ds(base, CH)], idx)
        pltpu.sync_copy(val_hs[v].at[pl.ds(base, CH)], buf_m)
        pltpu.sync_copy(buf_m, acc_m.at[idx], add=True)
        return 0

      lax.fori_loop(0, NCHUNK, sbody, 0)
      plsc.subcore_barrier()

      def wbody(j, _, v=v):
        c = sid + NS * j

        @pl.when(c < NZC)
        def _():
          pltpu.sync_copy(acc_m.at[pl.ds(c * ZR, ZR)],
                          out_hs[v].at[cid, pl.ds(c * ZR, ZR)])
        return 0

      lax.fori_loop(0, (NZC + NS - 1) // NS, wbody, 0)
      if v + 1 < nv:
        plsc.subcore_barrier()

  outs = k(row, z_m, *vals)
  if nv == 1 and not isinstance(outs, (list, tuple)):
    outs = (outs,)
  return outs


BLK_E = 2000
BLK_N = 1000


def _full(shape):
  return pl.BlockSpec(shape, lambda i: tuple(0 for _ in shape))


def _init_body(x, wp, bp, wa, wb, be1, h_o, a_o, b_o):
  h = jax.nn.silu(jnp.dot(x[...], wp[...], preferred_element_type=_f32)
                  + bp[...])
  h_o[...] = h
  a_o[...] = jnp.dot(h, wa[...], preferred_element_type=_f32) + be1[...]
  b_o[...] = jnp.dot(h, wb[...], preferred_element_type=_f32)


def _tc_init(x, wp, bp, wa, wb, be1):
  return pl.pallas_call(
      _init_body,
      grid=(N // BLK_N,),
      in_specs=[
          pl.BlockSpec((BLK_N, D), lambda i: (i, 0)),
          _full((D, D)), _full((1, D)), _full((D, D)), _full((D, D)),
          _full((1, D)),
      ],
      out_specs=[
          pl.BlockSpec((BLK_N, D), lambda i: (i, 0)),
          pl.BlockSpec((BLK_N, D), lambda i: (i, 0)),
          pl.BlockSpec((BLK_N, D), lambda i: (i, 0)),
      ],
      out_shape=[jax.ShapeDtypeStruct((N, D), _f32)] * 3,
      compiler_params=pltpu.CompilerParams(
          dimension_semantics=("parallel",)),
  )(x, wp, bp, wa, wb, be1)


def _edge_body_coord(ga, gb, pr, pc, w1l, we2, be2, wc1, bc1, wc2r, bc2v,
                     m2_o, ext_o):
  diff = pr[...] - pc[...]
  dist = jnp.sum(diff * diff, axis=-1, keepdims=True)
  m1 = jax.nn.silu(ga[...] + gb[...] + dist * w1l[...])
  m2 = jax.nn.silu(jnp.dot(m1, we2[...], preferred_element_type=_f32)
                   + be2[...])
  m2_o[...] = m2
  c1 = jax.nn.silu(jnp.dot(m2, wc1[...], preferred_element_type=_f32)
                   + bc1[...])
  cw = jnp.sum(c1 * wc2r[...], axis=-1, keepdims=True) + bc2v[...][:, 0:1]
  colid = lax.broadcasted_iota(jnp.int32, (BLK_E, D), 1)
  ext_o[...] = jnp.where(colid == 2, 1.0, diff * cw)


def _tc_edge_coord(ga, gb, pr, pc, w1l, we2, be2, wc1, bc1, wc2r, bc2v):
  return pl.pallas_call(
      _edge_body_coord,
      grid=(E // BLK_E,),
      in_specs=[
          pl.BlockSpec((BLK_E, D), lambda i: (i, 0)),
          pl.BlockSpec((BLK_E, D), lambda i: (i, 0)),
          pl.BlockSpec((BLK_E, D), lambda i: (i, 0)),
          pl.BlockSpec((BLK_E, D), lambda i: (i, 0)),
          _full((1, D)), _full((D, D)), _full((1, D)),
          _full((D, D)), _full((1, D)), _full((1, D)), _full((1, D)),
      ],
      out_specs=[
          pl.BlockSpec((BLK_E, D), lambda i: (i, 0)),
          pl.BlockSpec((BLK_E, D), lambda i: (i, 0)),
      ],
      out_shape=[
          jax.ShapeDtypeStruct((E, D), _f32),
          jax.ShapeDtypeStruct((E, D), _f32),
      ],
      compiler_params=pltpu.CompilerParams(
          dimension_semantics=("parallel",)),
  )(ga, gb, pr, pc, w1l, we2, be2, wc1, bc1, wc2r, bc2v)


def _edge_body_plain(ga, gb, pr, pc, w1l, we2, be2, m2_o):
  diff = pr[...] - pc[...]
  dist = jnp.sum(diff * diff, axis=-1, keepdims=True)
  m1 = jax.nn.silu(ga[...] + gb[...] + dist * w1l[...])
  m2_o[...] = jax.nn.silu(jnp.dot(m1, we2[...], preferred_element_type=_f32)
                          + be2[...])


def _tc_edge_plain(ga, gb, pr, pc, w1l, we2, be2):
  return pl.pallas_call(
      _edge_body_plain,
      grid=(E // BLK_E,),
      in_specs=[
          pl.BlockSpec((BLK_E, D), lambda i: (i, 0)),
          pl.BlockSpec((BLK_E, D), lambda i: (i, 0)),
          pl.BlockSpec((BLK_E, D), lambda i: (i, 0)),
          pl.BlockSpec((BLK_E, D), lambda i: (i, 0)),
          _full((1, D)), _full((D, D)), _full((1, D)),
      ],
      out_specs=pl.BlockSpec((BLK_E, D), lambda i: (i, 0)),
      out_shape=jax.ShapeDtypeStruct((E, D), _f32),
      compiler_params=pltpu.CompilerParams(
          dimension_semantics=("parallel",)),
  )(ga, gb, pr, pc, w1l, we2, be2)


def _node_mlp(h, msg, wn1a, wn1b, bn1, wn2, bn2, gam, bet):
  t = jax.nn.silu(jnp.dot(h, wn1a[...], preferred_element_type=_f32)
                  + jnp.dot(msg, wn1b[...], preferred_element_type=_f32)
                  + bn1[...])
  hn = jnp.dot(t, wn2[...], preferred_element_type=_f32) + bn2[...] + h
  mu = jnp.mean(hn, axis=-1, keepdims=True)
  var = jnp.mean((hn - mu) ** 2, axis=-1, keepdims=True)
  return (hn - mu) * lax.rsqrt(var + 1e-5) * gam[...] + bet[...]


def _node_body_mid(h, pm, pe, posp, wn1a, wn1b, bn1, wn2, bn2, gam, bet,
                   wa_n, wb_n, be1_n, h_o, a_o, b_o, posp_o):
  msg = pm[0] + pm[1]
  ho = _node_mlp(h[...], msg, wn1a, wn1b, bn1, wn2, bn2, gam, bet)
  h_o[...] = ho
  a_o[...] = jnp.dot(ho, wa_n[...], preferred_element_type=_f32) + be1_n[...]
  b_o[...] = jnp.dot(ho, wb_n[...], preferred_element_type=_f32)
  es = pe[0] + pe[1]
  cnt = jnp.maximum(es[:, 2:3], 1.0)
  colid = lax.broadcasted_iota(jnp.int32, (BLK_N, D), 1)
  posp_o[...] = posp[...] + jnp.where(colid < 2, es / cnt, 0.0)


def _tc_node_mid(h, pm, pe, posp, wn1a, wn1b, bn1, wn2, bn2, gam, bet,
                 wa_n, wb_n, be1_n):
  return pl.pallas_call(
      _node_body_mid,
      grid=(N // BLK_N,),
      in_specs=[
          pl.BlockSpec((BLK_N, D), lambda i: (i, 0)),
          pl.BlockSpec((NC, BLK_N, D), lambda i: (0, i, 0)),
          pl.BlockSpec((NC, BLK_N, D), lambda i: (0, i, 0)),
          pl.BlockSpec((BLK_N, D), lambda i: (i, 0)),
          _full((D, D)), _full((D, D)), _full((1, D)),
          _full((D, D)), _full((1, D)), _full((1, D)), _full((1, D)),
          _full((D, D)), _full((D, D)), _full((1, D)),
      ],
      out_specs=[
          pl.BlockSpec((BLK_N, D), lambda i: (i, 0)),
          pl.BlockSpec((BLK_N, D), lambda i: (i, 0)),
          pl.BlockSpec((BLK_N, D), lambda i: (i, 0)),
          pl.BlockSpec((BLK_N, D), lambda i: (i, 0)),
      ],
      out_shape=[
          jax.ShapeDtypeStruct((N, D), _f32),
          jax.ShapeDtypeStruct((N, D), _f32),
          jax.ShapeDtypeStruct((N, D), _f32),
          jax.ShapeDtypeStruct((N, D), _f32),
      ],
      compiler_params=pltpu.CompilerParams(
          dimension_semantics=("parallel",)),
  )(h, pm, pe, posp, wn1a, wn1b, bn1, wn2, bn2, gam, bet, wa_n, wb_n, be1_n)


def _node_body_last(h, pm, wn1a, wn1b, bn1, wn2, bn2, gam, bet, h_o):
  msg = pm[0] + pm[1]
  h_o[...] = _node_mlp(h[...], msg, wn1a, wn1b, bn1, wn2, bn2, gam, bet)


def _tc_node_last(h, pm, wn1a, wn1b, bn1, wn2, bn2, gam, bet):
  return pl.pallas_call(
      _node_body_last,
      grid=(N // BLK_N,),
      in_specs=[
          pl.BlockSpec((BLK_N, D), lambda i: (i, 0)),
          pl.BlockSpec((NC, BLK_N, D), lambda i: (0, i, 0)),
          _full((D, D)), _full((D, D)), _full((1, D)),
          _full((D, D)), _full((1, D)), _full((1, D)), _full((1, D)),
      ],
      out_specs=pl.BlockSpec((BLK_N, D), lambda i: (i, 0)),
      out_shape=jax.ShapeDtypeStruct((N, D), _f32),
      compiler_params=pltpu.CompilerParams(
          dimension_semantics=("parallel",)),
  )(h, pm, wn1a, wn1b, bn1, wn2, bn2, gam, bet)


def kernel(x, pos, edge_index, params):
  row = edge_index[0]
  col = edge_index[1]
  posp = jnp.pad(pos, ((0, 0), (0, D - 2)))
  z_m = jnp.zeros((ZR, D), _f32)

  def r1(v):
    return v.reshape(1, D)

  layers = params['layers']
  lp0 = layers[0]
  h, a_t, b_t = _tc_init(
      x, params['proj']['W'], r1(params['proj']['b']),
      lp0['We1'][0:D], lp0['We1'][D:2 * D], r1(lp0['be1']))

  for i, lp in enumerate(layers):
    w1l = lp['We1'][2 * D:2 * D + 1]
    ga, gb, pr, pc = _sc_gather(row, col, a_t, b_t, posp)
    if i < 2:
      wc2r = lp['Wc2'].reshape(1, D)
      bc2v = jnp.broadcast_to(lp['bc2'].reshape(1, 1), (1, D))
      m2, ext = _tc_edge_coord(ga, gb, pr, pc, w1l, lp['We2'], r1(lp['be2']),
                               lp['Wc1'], r1(lp['bc1']), wc2r, bc2v)
      pm, pe = _sc_scatter(row, z_m, m2, ext)
      nxt = layers[i + 1]
      h, a_t, b_t, posp = _tc_node_mid(
          h, pm, pe, posp,
          lp['Wn1'][0:D], lp['Wn1'][D:2 * D], r1(lp['bn1']),
          lp['Wn2'], r1(lp['bn2']), r1(lp['gamma']), r1(lp['beta']),
          nxt['We1'][0:D], nxt['We1'][D:2 * D], r1(nxt['be1']))
    else:
      m2 = _tc_edge_plain(ga, gb, pr, pc, w1l, lp['We2'], r1(lp['be2']))
      pm, = _sc_scatter(row, z_m, m2)
      h = _tc_node_last(
          h, pm,
          lp['Wn1'][0:D], lp['Wn1'][D:2 * D], r1(lp['bn1']),
          lp['Wn2'], r1(lp['bn2']), r1(lp['gamma']), r1(lp['beta']))
  return h


# speedup vs baseline: 3.9604x; 1.4175x over previous
import functools

import jax
import jax.numpy as jnp
from jax import lax
from jax.experimental import pallas as pl
from jax.experimental.pallas import tpu as pltpu
from jax.experimental.pallas import tpu_sc as plsc

N = 10000
E = 320000
D = 128
P = 8

NC = 2
NS = 16
NW = NC * NS
EPW = E // NW
CH = 80
NCHUNK = EPW // CH
ZR = 400
NZC = N // ZR

@functools.cache
def _sc_mesh():
  return plsc.VectorSubcoreMesh(
      core_axis_name="c", subcore_axis_name="s", num_cores=NC, num_subcores=NS)


_f32 = jnp.float32


def _sw_pipe(nch, fa, wa, fb, wb):
  fa(0, 0)
  fa(1, 1)
  wa(0)
  fb(0, 0)

  def pair(j, _):
    i = 2 * j
    wb(0)
    fa(i + 2, 0)
    wa(1)
    fb(i + 1, 1)
    wb(1)
    fa(i + 3, 1)
    wa(0)
    fb(i + 2, 0)
    return 0

  lax.fori_loop(0, (nch - 2) // 2, pair, 0)
  if nch % 2:
    wb(0)
    fa(nch - 1, 0)
    wa(1)
    fb(nch - 2, 1)
    wa(0)
    fb(nch - 1, 0)
    wb(1)
    wb(0)
  else:
    wa(1)
    fb(nch - 1, 1)
    wb(0)
    wb(1)


def _sc_gather(row, col, a_t, b_t, posp):
  @functools.partial(
      pl.kernel,
      out_type=[
          jax.ShapeDtypeStruct((E, D), _f32),
          jax.ShapeDtypeStruct((E, D), _f32),
          jax.ShapeDtypeStruct((E, D), _f32),
          jax.ShapeDtypeStruct((E, D), _f32),
      ],
      mesh=_sc_mesh(),
      scratch_types=[
          pltpu.VMEM((EPW,), jnp.int32),
          pltpu.VMEM((EPW,), jnp.int32),
          [pltpu.VMEM((CH, D), _f32) for _ in range(8)],
          [pltpu.SemaphoreType.DMA for _ in range(2)],
          [pltpu.SemaphoreType.DMA for _ in range(2)],
      ],
  )
  def k(row_h, col_h, a_h, b_h, p_h, ga_h, gb_h, pr_h, pc_h,
        idx_r, idx_c, bufs, sg, sw):
    wid = lax.axis_index("s") * NC + lax.axis_index("c")
    ebase = wid * EPW
    pltpu.sync_copy(row_h.at[pl.ds(ebase, EPW)], idx_r)
    pltpu.sync_copy(col_h.at[pl.ds(ebase, EPW)], idx_c)
    outs = (ga_h, gb_h, pr_h, pc_h)

    def fire(i, s):
      o = i * CH
      srcs = (a_h.at[idx_r.at[pl.ds(o, CH)]],
              b_h.at[idx_c.at[pl.ds(o, CH)]],
              p_h.at[idx_r.at[pl.ds(o, CH)]],
              p_h.at[idx_c.at[pl.ds(o, CH)]])
      for t in range(4):
        pltpu.async_copy(srcs[t], bufs[4 * s + t], sg[s])

    def wait_g(s):
      for t in range(4):
        pltpu.make_async_copy(outs[t].at[pl.ds(0, CH)], bufs[4 * s + t],
                              sg[s]).wait()

    def writeback(i, s):
      base = ebase + i * CH
      for t in range(4):
        pltpu.async_copy(bufs[4 * s + t], outs[t].at[pl.ds(base, CH)], sw[s])

    def wait_w(s):
      for t in range(4):
        pltpu.make_async_copy(bufs[4 * s + t], outs[t].at[pl.ds(0, CH)],
                              sw[s]).wait()

    _sw_pipe(NCHUNK, fire, wait_g, writeback, wait_w)

  return k(row, col, a_t, b_t, posp)


def _zero_acc(sid, zm_h, acc_m):
  def zbody(j, _):
    c = sid + NS * j

    @pl.when(c < NZC)
    def _():
      pltpu.sync_copy(zm_h, acc_m.at[pl.ds(c * ZR, ZR)])
    return 0

  lax.fori_loop(0, (NZC + NS - 1) // NS, zbody, 0)


def _scat_pipeline(nch, ebase, row_h, val_h, acc_m, ix, vb, sl, ss):
  def fire_load(i, s):
    base = ebase + i * CH
    pltpu.async_copy(row_h.at[pl.ds(base, CH)], ix[s], sl[s])
    pltpu.async_copy(val_h.at[pl.ds(base, CH)], vb[s], sl[s])

  def wait_load(s):
    pltpu.make_async_copy(row_h.at[pl.ds(0, CH)], ix[s], sl[s]).wait()
    pltpu.make_async_copy(val_h.at[pl.ds(0, CH)], vb[s], sl[s]).wait()

  def fire_scat(i, s):
    pltpu.async_copy(vb[s], acc_m.at[ix[s]], ss[s], add=True)

  def wait_scat(s):
    pltpu.make_async_copy(vb[s], acc_m.at[pl.ds(0, CH)], ss[s]).wait()

  _sw_pipe(nch, fire_load, wait_load, fire_scat, wait_scat)


_SCAT_SCRATCH = [
    [pltpu.VMEM((CH,), jnp.int32) for _ in range(2)],
    [pltpu.VMEM((CH, D), _f32) for _ in range(2)],
    [pltpu.SemaphoreType.DMA for _ in range(2)],
    [pltpu.SemaphoreType.DMA for _ in range(2)],
    pltpu.VMEM_SHARED((N, D), _f32),
]

EPS = E // NS
NCHS = EPS // CH


def _sc_scatter_dual(row, z_m, m2, ext):
  @functools.partial(
      pl.kernel,
      out_type=[
          jax.ShapeDtypeStruct((N, D), _f32),
          jax.ShapeDtypeStruct((N, D), _f32),
      ],
      mesh=_sc_mesh(),
      scratch_types=_SCAT_SCRATCH,
  )
  def k(row_h, zm_h, m2_h, ext_h, pm_h, pe_h, ix, vb, sl, ss, acc_m):
    cid = lax.axis_index("c")
    sid = lax.axis_index("s")
    _zero_acc(sid, zm_h, acc_m)
    plsc.subcore_barrier()
    ebase = sid * EPS

    @pl.when(cid == 0)
    def _():
      _scat_pipeline(NCHS, ebase, row_h, m2_h, acc_m, ix, vb, sl, ss)

    @pl.when(cid == 1)
    def _():
      _scat_pipeline(NCHS, ebase, row_h, ext_h, acc_m, ix, vb, sl, ss)

    plsc.subcore_barrier()

    def wbody(j, _):
      c = sid + NS * j

      @pl.when(c < NZC)
      def _():
        @pl.when(cid == 0)
        def _():
          pltpu.sync_copy(acc_m.at[pl.ds(c * ZR, ZR)],
                          pm_h.at[pl.ds(c * ZR, ZR)])

        @pl.when(cid == 1)
        def _():
          pltpu.sync_copy(acc_m.at[pl.ds(c * ZR, ZR)],
                          pe_h.at[pl.ds(c * ZR, ZR)])
      return 0

    lax.fori_loop(0, (NZC + NS - 1) // NS, wbody, 0)

  return k(row, z_m, m2, ext)


def _sc_scatter_part(row, z_m, m2):
  @functools.partial(
      pl.kernel,
      out_type=jax.ShapeDtypeStruct((NC, N, D), _f32),
      mesh=_sc_mesh(),
      scratch_types=_SCAT_SCRATCH,
  )
  def k(row_h, zm_h, m2_h, pm_h, ix, vb, sl, ss, acc_m):
    cid = lax.axis_index("c")
    sid = lax.axis_index("s")
    _zero_acc(sid, zm_h, acc_m)
    plsc.subcore_barrier()
    _scat_pipeline(NCHUNK, (sid * NC + cid) * EPW, row_h, m2_h, acc_m,
                   ix, vb, sl, ss)
    plsc.subcore_barrier()

    def wbody(j, _):
      c = sid + NS * j

      @pl.when(c < NZC)
      def _():
        pltpu.sync_copy(acc_m.at[pl.ds(c * ZR, ZR)],
                        pm_h.at[cid, pl.ds(c * ZR, ZR)])
      return 0

    lax.fori_loop(0, (NZC + NS - 1) // NS, wbody, 0)

  return k(row, z_m, m2)


BLK_E = 2000
BLK_N = 1000


def _full(shape):
  return pl.BlockSpec(shape, lambda i: tuple(0 for _ in shape))


def _init_body(x, wp, bp, wa, wb, be1, h_o, a_o, b_o):
  h = jax.nn.silu(jnp.dot(x[...], wp[...], preferred_element_type=_f32)
                  + bp[...])
  h_o[...] = h
  a_o[...] = jnp.dot(h, wa[...], preferred_element_type=_f32) + be1[...]
  b_o[...] = jnp.dot(h, wb[...], preferred_element_type=_f32)


def _tc_init(x, wp, bp, wa, wb, be1):
  return pl.pallas_call(
      _init_body,
      grid=(N // BLK_N,),
      in_specs=[
          pl.BlockSpec((BLK_N, D), lambda i: (i, 0)),
          _full((D, D)), _full((1, D)), _full((D, D)), _full((D, D)),
          _full((1, D)),
      ],
      out_specs=[
          pl.BlockSpec((BLK_N, D), lambda i: (i, 0)),
          pl.BlockSpec((BLK_N, D), lambda i: (i, 0)),
          pl.BlockSpec((BLK_N, D), lambda i: (i, 0)),
      ],
      out_shape=[jax.ShapeDtypeStruct((N, D), _f32)] * 3,
      compiler_params=pltpu.CompilerParams(
          dimension_semantics=("parallel",)),
  )(x, wp, bp, wa, wb, be1)


def _edge_body_coord(ga, gb, pr, pc, w1l, we2, be2, wc1, bc1, wc2r, bc2v,
                     m2_o, ext_o):
  diff = pr[...] - pc[...]
  dist = jnp.sum(diff * diff, axis=-1, keepdims=True)
  m1 = jax.nn.silu(ga[...] + gb[...] + dist * w1l[...])
  m2 = jax.nn.silu(jnp.dot(m1, we2[...], preferred_element_type=_f32)
                   + be2[...])
  m2_o[...] = m2
  c1 = jax.nn.silu(jnp.dot(m2, wc1[...], preferred_element_type=_f32)
                   + bc1[...])
  cw = jnp.sum(c1 * wc2r[...], axis=-1, keepdims=True) + bc2v[...][:, 0:1]
  colid = lax.broadcasted_iota(jnp.int32, (BLK_E, D), 1)
  ext_o[...] = jnp.where(colid == 2, 1.0, diff * cw)


def _tc_edge_coord(ga, gb, pr, pc, w1l, we2, be2, wc1, bc1, wc2r, bc2v):
  return pl.pallas_call(
      _edge_body_coord,
      grid=(E // BLK_E,),
      in_specs=[
          pl.BlockSpec((BLK_E, D), lambda i: (i, 0)),
          pl.BlockSpec((BLK_E, D), lambda i: (i, 0)),
          pl.BlockSpec((BLK_E, D), lambda i: (i, 0)),
          pl.BlockSpec((BLK_E, D), lambda i: (i, 0)),
          _full((1, D)), _full((D, D)), _full((1, D)),
          _full((D, D)), _full((1, D)), _full((1, D)), _full((1, D)),
      ],
      out_specs=[
          pl.BlockSpec((BLK_E, D), lambda i: (i, 0)),
          pl.BlockSpec((BLK_E, D), lambda i: (i, 0)),
      ],
      out_shape=[
          jax.ShapeDtypeStruct((E, D), _f32),
          jax.ShapeDtypeStruct((E, D), _f32),
      ],
      compiler_params=pltpu.CompilerParams(
          dimension_semantics=("parallel",)),
  )(ga, gb, pr, pc, w1l, we2, be2, wc1, bc1, wc2r, bc2v)


def _edge_body_plain(ga, gb, pr, pc, w1l, we2, be2, m2_o):
  diff = pr[...] - pc[...]
  dist = jnp.sum(diff * diff, axis=-1, keepdims=True)
  m1 = jax.nn.silu(ga[...] + gb[...] + dist * w1l[...])
  m2_o[...] = jax.nn.silu(jnp.dot(m1, we2[...], preferred_element_type=_f32)
                          + be2[...])


def _tc_edge_plain(ga, gb, pr, pc, w1l, we2, be2):
  return pl.pallas_call(
      _edge_body_plain,
      grid=(E // BLK_E,),
      in_specs=[
          pl.BlockSpec((BLK_E, D), lambda i: (i, 0)),
          pl.BlockSpec((BLK_E, D), lambda i: (i, 0)),
          pl.BlockSpec((BLK_E, D), lambda i: (i, 0)),
          pl.BlockSpec((BLK_E, D), lambda i: (i, 0)),
          _full((1, D)), _full((D, D)), _full((1, D)),
      ],
      out_specs=pl.BlockSpec((BLK_E, D), lambda i: (i, 0)),
      out_shape=jax.ShapeDtypeStruct((E, D), _f32),
      compiler_params=pltpu.CompilerParams(
          dimension_semantics=("parallel",)),
  )(ga, gb, pr, pc, w1l, we2, be2)


def _node_mlp(h, msg, wn1a, wn1b, bn1, wn2, bn2, gam, bet):
  t = jax.nn.silu(jnp.dot(h, wn1a[...], preferred_element_type=_f32)
                  + jnp.dot(msg, wn1b[...], preferred_element_type=_f32)
                  + bn1[...])
  hn = jnp.dot(t, wn2[...], preferred_element_type=_f32) + bn2[...] + h
  mu = jnp.mean(hn, axis=-1, keepdims=True)
  var = jnp.mean((hn - mu) ** 2, axis=-1, keepdims=True)
  return (hn - mu) * lax.rsqrt(var + 1e-5) * gam[...] + bet[...]


def _node_body_mid(h, pm, pe, posp, wn1a, wn1b, bn1, wn2, bn2, gam, bet,
                   wa_n, wb_n, be1_n, h_o, a_o, b_o, posp_o):
  msg = pm[...]
  ho = _node_mlp(h[...], msg, wn1a, wn1b, bn1, wn2, bn2, gam, bet)
  h_o[...] = ho
  a_o[...] = jnp.dot(ho, wa_n[...], preferred_element_type=_f32) + be1_n[...]
  b_o[...] = jnp.dot(ho, wb_n[...], preferred_element_type=_f32)
  es = pe[...]
  cnt = jnp.maximum(es[:, 2:3], 1.0)
  colid = lax.broadcasted_iota(jnp.int32, (BLK_N, D), 1)
  posp_o[...] = posp[...] + jnp.where(colid < 2, es / cnt, 0.0)


def _tc_node_mid(h, pm, pe, posp, wn1a, wn1b, bn1, wn2, bn2, gam, bet,
                 wa_n, wb_n, be1_n):
  return pl.pallas_call(
      _node_body_mid,
      grid=(N // BLK_N,),
      in_specs=[
          pl.BlockSpec((BLK_N, D), lambda i: (i, 0)),
          pl.BlockSpec((BLK_N, D), lambda i: (i, 0)),
          pl.BlockSpec((BLK_N, D), lambda i: (i, 0)),
          pl.BlockSpec((BLK_N, D), lambda i: (i, 0)),
          _full((D, D)), _full((D, D)), _full((1, D)),
          _full((D, D)), _full((1, D)), _full((1, D)), _full((1, D)),
          _full((D, D)), _full((D, D)), _full((1, D)),
      ],
      out_specs=[
          pl.BlockSpec((BLK_N, D), lambda i: (i, 0)),
          pl.BlockSpec((BLK_N, D), lambda i: (i, 0)),
          pl.BlockSpec((BLK_N, D), lambda i: (i, 0)),
          pl.BlockSpec((BLK_N, D), lambda i: (i, 0)),
      ],
      out_shape=[
          jax.ShapeDtypeStruct((N, D), _f32),
          jax.ShapeDtypeStruct((N, D), _f32),
          jax.ShapeDtypeStruct((N, D), _f32),
          jax.ShapeDtypeStruct((N, D), _f32),
      ],
      compiler_params=pltpu.CompilerParams(
          dimension_semantics=("parallel",)),
  )(h, pm, pe, posp, wn1a, wn1b, bn1, wn2, bn2, gam, bet, wa_n, wb_n, be1_n)


def _node_body_last(h, pm, wn1a, wn1b, bn1, wn2, bn2, gam, bet, h_o):
  msg = pm[0] + pm[1]
  h_o[...] = _node_mlp(h[...], msg, wn1a, wn1b, bn1, wn2, bn2, gam, bet)


def _tc_node_last(h, pm, wn1a, wn1b, bn1, wn2, bn2, gam, bet):
  return pl.pallas_call(
      _node_body_last,
      grid=(N // BLK_N,),
      in_specs=[
          pl.BlockSpec((BLK_N, D), lambda i: (i, 0)),
          pl.BlockSpec((NC, BLK_N, D), lambda i: (0, i, 0)),
          _full((D, D)), _full((D, D)), _full((1, D)),
          _full((D, D)), _full((1, D)), _full((1, D)), _full((1, D)),
      ],
      out_specs=pl.BlockSpec((BLK_N, D), lambda i: (i, 0)),
      out_shape=jax.ShapeDtypeStruct((N, D), _f32),
      compiler_params=pltpu.CompilerParams(
          dimension_semantics=("parallel",)),
  )(h, pm, wn1a, wn1b, bn1, wn2, bn2, gam, bet)


def kernel(x, pos, edge_index, params):
  row = edge_index[0]
  col = edge_index[1]
  posp = jnp.pad(pos, ((0, 0), (0, D - 2)))
  z_m = jnp.zeros((ZR, D), _f32)

  def r1(v):
    return v.reshape(1, D)

  layers = params['layers']
  lp0 = layers[0]
  h, a_t, b_t = _tc_init(
      x, params['proj']['W'], r1(params['proj']['b']),
      lp0['We1'][0:D], lp0['We1'][D:2 * D], r1(lp0['be1']))

  for i, lp in enumerate(layers):
    w1l = lp['We1'][2 * D:2 * D + 1]
    ga, gb, pr, pc = _sc_gather(row, col, a_t, b_t, posp)
    if i < 2:
      wc2r = lp['Wc2'].reshape(1, D)
      bc2v = jnp.broadcast_to(lp['bc2'].reshape(1, 1), (1, D))
      m2, ext = _tc_edge_coord(ga, gb, pr, pc, w1l, lp['We2'], r1(lp['be2']),
                               lp['Wc1'], r1(lp['bc1']), wc2r, bc2v)
      pm, pe = _sc_scatter_dual(row, z_m, m2, ext)
      nxt = layers[i + 1]
      h, a_t, b_t, posp = _tc_node_mid(
          h, pm, pe, posp,
          lp['Wn1'][0:D], lp['Wn1'][D:2 * D], r1(lp['bn1']),
          lp['Wn2'], r1(lp['bn2']), r1(lp['gamma']), r1(lp['beta']),
          nxt['We1'][0:D], nxt['We1'][D:2 * D], r1(nxt['be1']))
    else:
      m2 = _tc_edge_plain(ga, gb, pr, pc, w1l, lp['We2'], r1(lp['be2']))
      pm = _sc_scatter_part(row, z_m, m2)
      h = _tc_node_last(
          h, pm,
          lp['Wn1'][0:D], lp['Wn1'][D:2 * D], r1(lp['bn1']),
          lp['Wn2'], r1(lp['bn2']), r1(lp['gamma']), r1(lp['beta']))
  return h


# speedup vs baseline: 4.1712x; 1.0532x over previous
import functools

import jax
import jax.numpy as jnp
from jax import lax
from jax.experimental import pallas as pl
from jax.experimental.pallas import tpu as pltpu
from jax.experimental.pallas import tpu_sc as plsc

N = 10000
E = 320000
D = 128
P = 8

NC = 2
NS = 16
NW = NC * NS
EPW = E // NW
CH = 80
NCHUNK = EPW // CH
ZR = 400
NZC = N // ZR

@functools.cache
def _sc_mesh():
  return plsc.VectorSubcoreMesh(
      core_axis_name="c", subcore_axis_name="s", num_cores=NC, num_subcores=NS)


_f32 = jnp.float32


def _sw_pipe(nch, fa, wa, fb, wb):
  fa(0, 0)
  fa(1, 1)
  wa(0)
  fb(0, 0)

  def pair(j, _):
    i = 2 * j
    wb(0)
    fa(i + 2, 0)
    wa(1)
    fb(i + 1, 1)
    wb(1)
    fa(i + 3, 1)
    wa(0)
    fb(i + 2, 0)
    return 0

  lax.fori_loop(0, (nch - 2) // 2, pair, 0)
  if nch % 2:
    wb(0)
    fa(nch - 1, 0)
    wa(1)
    fb(nch - 2, 1)
    wa(0)
    fb(nch - 1, 0)
    wb(1)
    wb(0)
  else:
    wa(1)
    fb(nch - 1, 1)
    wb(0)
    wb(1)


def _sc_gather(row, col, a_t, b_t, posq):
  @functools.partial(
      pl.kernel,
      out_type=[jax.ShapeDtypeStruct((E, D), _f32)] * 2
      + [jax.ShapeDtypeStruct((E, P), _f32)] * 2,
      mesh=_sc_mesh(),
      scratch_types=[
          pltpu.VMEM((EPW,), jnp.int32),
          pltpu.VMEM((EPW,), jnp.int32),
          [pltpu.VMEM((CH, D), _f32) for _ in range(4)],
          [pltpu.VMEM((CH, P), _f32) for _ in range(4)],
          [pltpu.SemaphoreType.DMA for _ in range(2)],
          [pltpu.SemaphoreType.DMA for _ in range(2)],
      ],
      compiler_params=pltpu.CompilerParams(use_tc_tiling_on_sc=False),
  )
  def k(row_h, col_h, a_h, b_h, pq_h, ga_h, gb_h, pr_h, pc_h,
        idx_r, idx_c, bufs, pbufs, sg, sw):
    wid = lax.axis_index("s") * NC + lax.axis_index("c")
    ebase = wid * EPW
    pltpu.sync_copy(row_h.at[pl.ds(ebase, EPW)], idx_r)
    pltpu.sync_copy(col_h.at[pl.ds(ebase, EPW)], idx_c)
    outs = (ga_h, gb_h)
    pouts = (pr_h, pc_h)

    def fire(i, s):
      o = i * CH
      pltpu.async_copy(a_h.at[idx_r.at[pl.ds(o, CH)]], bufs[2 * s], sg[s])
      pltpu.async_copy(b_h.at[idx_c.at[pl.ds(o, CH)]], bufs[2 * s + 1], sg[s])
      pltpu.async_copy(pq_h.at[idx_r.at[pl.ds(o, CH)]], pbufs[2 * s], sg[s])
      pltpu.async_copy(pq_h.at[idx_c.at[pl.ds(o, CH)]], pbufs[2 * s + 1],
                       sg[s])

    def wait_g(s):
      for t in range(2):
        pltpu.make_async_copy(outs[t].at[pl.ds(0, CH)], bufs[2 * s + t],
                              sg[s]).wait()
        pltpu.make_async_copy(pouts[t].at[pl.ds(0, CH)], pbufs[2 * s + t],
                              sg[s]).wait()

    def writeback(i, s):
      base = ebase + i * CH
      for t in range(2):
        pltpu.async_copy(bufs[2 * s + t], outs[t].at[pl.ds(base, CH)], sw[s])
        pltpu.async_copy(pbufs[2 * s + t], pouts[t].at[pl.ds(base, CH)],
                         sw[s])

    def wait_w(s):
      for t in range(2):
        pltpu.make_async_copy(bufs[2 * s + t], outs[t].at[pl.ds(0, CH)],
                              sw[s]).wait()
        pltpu.make_async_copy(pbufs[2 * s + t], pouts[t].at[pl.ds(0, CH)],
                              sw[s]).wait()

    _sw_pipe(NCHUNK, fire, wait_g, writeback, wait_w)

  return k(row, col, a_t, b_t, posq)


def _zero_acc(sid, zm_h, acc_m):
  def zbody(j, _):
    c = sid + NS * j

    @pl.when(c < NZC)
    def _():
      pltpu.sync_copy(zm_h, acc_m.at[pl.ds(c * ZR, ZR)])
    return 0

  lax.fori_loop(0, (NZC + NS - 1) // NS, zbody, 0)


def _scat_pipeline(nch, ebase, row_h, val_h, acc_m, ix, vb, sl, ss):
  def fire_load(i, s):
    base = ebase + i * CH
    pltpu.async_copy(row_h.at[pl.ds(base, CH)], ix[s], sl[s])
    pltpu.async_copy(val_h.at[pl.ds(base, CH)], vb[s], sl[s])

  def wait_load(s):
    pltpu.make_async_copy(row_h.at[pl.ds(0, CH)], ix[s], sl[s]).wait()
    pltpu.make_async_copy(val_h.at[pl.ds(0, CH)], vb[s], sl[s]).wait()

  def fire_scat(i, s):
    pltpu.async_copy(vb[s], acc_m.at[ix[s]], ss[s], add=True)

  def wait_scat(s):
    pltpu.make_async_copy(vb[s], acc_m.at[pl.ds(0, CH)], ss[s]).wait()

  _sw_pipe(nch, fire_load, wait_load, fire_scat, wait_scat)


_SCAT_SCRATCH = [
    [pltpu.VMEM((CH,), jnp.int32) for _ in range(2)],
    [pltpu.VMEM((CH, D), _f32) for _ in range(2)],
    [pltpu.SemaphoreType.DMA for _ in range(2)],
    [pltpu.SemaphoreType.DMA for _ in range(2)],
    pltpu.VMEM_SHARED((N, D), _f32),
]


def _sc_scatter_dual(row, z_m, m2, ext):
  @functools.partial(
      pl.kernel,
      out_type=[
          jax.ShapeDtypeStruct((N, D), _f32),
          jax.ShapeDtypeStruct((N, D), _f32),
      ],
      mesh=_sc_mesh(),
      scratch_types=_SCAT_SCRATCH,
  )
  def k(row_h, zm_h, m2_h, ext_h, pm_h, pe_h, ix, vb, sl, ss, acc_m):
    cid = lax.axis_index("c")
    sid = lax.axis_index("s")

    def zbody(j, _):
      c = sid + NS * j

      @pl.when(c < NZC)
      def _():
        pltpu.sync_copy(zm_h, acc_m.at[pl.ds(c * ZR, ZR)])
      return 0

    lax.fori_loop(0, (NZC + NS - 1) // NS, zbody, 0)
    plsc.subcore_barrier()
    ebase = sid * EPS

    def mk(val_h):
      def fire_load(i, s):
        base = ebase + i * CH
        pltpu.async_copy(row_h.at[pl.ds(base, CH)], ix[s], sl[s])
        pltpu.async_copy(val_h.at[pl.ds(base, CH)], vb[s], sl[s])

      def wait_load(s):
        pltpu.make_async_copy(row_h.at[pl.ds(0, CH)], ix[s], sl[s]).wait()
        pltpu.make_async_copy(val_h.at[pl.ds(0, CH)], vb[s], sl[s]).wait()

      def fire_scat(i, s):
        pltpu.async_copy(vb[s], acc_m.at[ix[s]], ss[s], add=True)

      def wait_scat(s):
        pltpu.make_async_copy(vb[s], acc_m.at[pl.ds(0, CH)], ss[s]).wait()

      return fire_load, wait_load, fire_scat, wait_scat

    @pl.when(cid == 0)
    def _():
      _sw_pipe(NCHS, *mk(m2_h))

    @pl.when(cid == 1)
    def _():
      _sw_pipe(NCHS, *mk(ext_h))

    plsc.subcore_barrier()

    def wbody(j, _):
      c = sid + NS * j

      @pl.when(c < NZC)
      def _():
        @pl.when(cid == 0)
        def _():
          pltpu.sync_copy(acc_m.at[pl.ds(c * ZR, ZR)],
                          pm_h.at[pl.ds(c * ZR, ZR)])

        @pl.when(cid == 1)
        def _():
          pltpu.sync_copy(acc_m.at[pl.ds(c * ZR, ZR)],
                          pe_h.at[pl.ds(c * ZR, ZR)])
      return 0

    lax.fori_loop(0, (NZC + NS - 1) // NS, wbody, 0)

  return k(row, z_m, m2, ext)


EPS = E // NS
NCHS = EPS // CH


def _sc_scatter(row, z_m, z_e, m2, ext):
  with_ext = ext is not None
  scratch = list(_SCAT_SCRATCH)
  if with_ext:
    scratch += [
        [pltpu.VMEM((CH, P), _f32) for _ in range(2)],
        pltpu.VMEM_SHARED((N, P), _f32),
    ]
  out_type = [jax.ShapeDtypeStruct((NC, N, D), _f32)]
  if with_ext:
    out_type.append(jax.ShapeDtypeStruct((NC, N, P), _f32))

  @functools.partial(
      pl.kernel,
      out_type=out_type,
      mesh=_sc_mesh(),
      scratch_types=scratch,
      compiler_params=pltpu.CompilerParams(use_tc_tiling_on_sc=False),
  )
  def k(*args):
    if with_ext:
      (row_h, zm_h, ze_h, m2_h, ext_h, pm_h, pe_h,
       ix, vb, sl, ss, acc_m, eb, acc_e) = args
    else:
      row_h, zm_h, ze_h, m2_h, pm_h, ix, vb, sl, ss, acc_m = args
    cid = lax.axis_index("c")
    sid = lax.axis_index("s")
    ebase = (sid * NC + cid) * EPW

    def zbody(j, _):
      c = sid + NS * j

      @pl.when(c < NZC)
      def _():
        pltpu.sync_copy(zm_h, acc_m.at[pl.ds(c * ZR, ZR)])
        if with_ext:
          pltpu.sync_copy(ze_h, acc_e.at[pl.ds(c * ZR, ZR)])
      return 0

    lax.fori_loop(0, (NZC + NS - 1) // NS, zbody, 0)
    plsc.subcore_barrier()

    def fire_load(i, s):
      base = ebase + i * CH
      pltpu.async_copy(row_h.at[pl.ds(base, CH)], ix[s], sl[s])
      pltpu.async_copy(m2_h.at[pl.ds(base, CH)], vb[s], sl[s])
      if with_ext:
        pltpu.async_copy(ext_h.at[pl.ds(base, CH)], eb[s], sl[s])

    def wait_load(s):
      pltpu.make_async_copy(row_h.at[pl.ds(0, CH)], ix[s], sl[s]).wait()
      pltpu.make_async_copy(m2_h.at[pl.ds(0, CH)], vb[s], sl[s]).wait()
      if with_ext:
        pltpu.make_async_copy(ext_h.at[pl.ds(0, CH)], eb[s], sl[s]).wait()

    def fire_scat(i, s):
      pltpu.async_copy(vb[s], acc_m.at[ix[s]], ss[s], add=True)
      if with_ext:
        pltpu.async_copy(eb[s], acc_e.at[ix[s]], ss[s], add=True)

    def wait_scat(s):
      pltpu.make_async_copy(vb[s], acc_m.at[pl.ds(0, CH)], ss[s]).wait()
      if with_ext:
        pltpu.make_async_copy(eb[s], acc_e.at[pl.ds(0, CH)], ss[s]).wait()

    _sw_pipe(NCHUNK, fire_load, wait_load, fire_scat, wait_scat)
    plsc.subcore_barrier()

    def wbody(j, _):
      c = sid + NS * j

      @pl.when(c < NZC)
      def _():
        pltpu.sync_copy(acc_m.at[pl.ds(c * ZR, ZR)],
                        pm_h.at[cid, pl.ds(c * ZR, ZR)])
        if with_ext:
          pltpu.sync_copy(acc_e.at[pl.ds(c * ZR, ZR)],
                          pe_h.at[cid, pl.ds(c * ZR, ZR)])
      return 0

    lax.fori_loop(0, (NZC + NS - 1) // NS, wbody, 0)

  if with_ext:
    return k(row, z_m, z_e, m2, ext)
  res = k(row, z_m, z_e, m2)
  if isinstance(res, (list, tuple)):
    res = res[0]
  return res, None


BLK_E = 4000
BLK_N = 1000


def _full(shape):
  return pl.BlockSpec(shape, lambda i: tuple(0 for _ in shape))


def _init_body(x, wp, bp, wa, wb, be1, h_o, a_o, b_o):
  h = jax.nn.silu(jnp.dot(x[...], wp[...], preferred_element_type=_f32)
                  + bp[...])
  h_o[...] = h
  a_o[...] = jnp.dot(h, wa[...], preferred_element_type=_f32) + be1[...]
  b_o[...] = jnp.dot(h, wb[...], preferred_element_type=_f32)


def _tc_init(x, wp, bp, wa, wb, be1):
  return pl.pallas_call(
      _init_body,
      grid=(N // BLK_N,),
      in_specs=[
          pl.BlockSpec((BLK_N, D), lambda i: (i, 0)),
          _full((D, D)), _full((1, D)), _full((D, D)), _full((D, D)),
          _full((1, D)),
      ],
      out_specs=[
          pl.BlockSpec((BLK_N, D), lambda i: (i, 0)),
          pl.BlockSpec((BLK_N, D), lambda i: (i, 0)),
          pl.BlockSpec((BLK_N, D), lambda i: (i, 0)),
      ],
      out_shape=[jax.ShapeDtypeStruct((N, D), _f32)] * 3,
      compiler_params=pltpu.CompilerParams(
          dimension_semantics=("parallel",)),
  )(x, wp, bp, wa, wb, be1)


def _edge_body_coord(ga, gb, pr, pc, w1l, we2, be2, wc1, bc1, wc2r, bc2v,
                     m2_o, ext_o):
  dif = pr[...] - pc[...]
  dist = jnp.sum(dif * dif, axis=-1, keepdims=True)
  m1 = jax.nn.silu(ga[...] + gb[...] + dist * w1l[...])
  m2 = jax.nn.silu(jnp.dot(m1, we2[...], preferred_element_type=_f32)
                   + be2[...])
  m2_o[...] = m2
  c1 = jax.nn.silu(jnp.dot(m2, wc1[...], preferred_element_type=_f32)
                   + bc1[...])
  cw = jnp.sum(c1 * wc2r[...], axis=-1, keepdims=True) + bc2v[...][:, 0:1]
  dx = dif[:, 0:1]
  dy = dif[:, 1:2]
  colid = lax.broadcasted_iota(jnp.int32, (BLK_E, D), 1)
  ext_o[...] = jnp.where(
      colid == 0, dx * cw,
      jnp.where(colid == 1, dy * cw,
                jnp.where(colid == 2, 1.0, 0.0)))


def _tc_edge_coord(ga, gb, pr, pc, w1l, we2, be2, wc1, bc1, wc2r, bc2v):
  return pl.pallas_call(
      _edge_body_coord,
      grid=(E // BLK_E,),
      in_specs=[
          pl.BlockSpec((BLK_E, D), lambda i: (i, 0)),
          pl.BlockSpec((BLK_E, D), lambda i: (i, 0)),
          pl.BlockSpec((BLK_E, P), lambda i: (i, 0)),
          pl.BlockSpec((BLK_E, P), lambda i: (i, 0)),
          _full((1, D)), _full((D, D)), _full((1, D)),
          _full((D, D)), _full((1, D)), _full((1, D)), _full((1, D)),
      ],
      out_specs=[
          pl.BlockSpec((BLK_E, D), lambda i: (i, 0)),
          pl.BlockSpec((BLK_E, D), lambda i: (i, 0)),
      ],
      out_shape=[
          jax.ShapeDtypeStruct((E, D), _f32),
          jax.ShapeDtypeStruct((E, D), _f32),
      ],
      compiler_params=pltpu.CompilerParams(
          dimension_semantics=("parallel",)),
  )(ga, gb, pr, pc, w1l, we2, be2, wc1, bc1, wc2r, bc2v)


def _edge_body_plain(ga, gb, pr, pc, w1l, we2, be2, m2_o):
  dif = pr[...] - pc[...]
  dist = jnp.sum(dif * dif, axis=-1, keepdims=True)
  m1 = jax.nn.silu(ga[...] + gb[...] + dist * w1l[...])
  m2_o[...] = jax.nn.silu(jnp.dot(m1, we2[...], preferred_element_type=_f32)
                          + be2[...])


def _tc_edge_plain(ga, gb, pr, pc, w1l, we2, be2):
  return pl.pallas_call(
      _edge_body_plain,
      grid=(E // BLK_E,),
      in_specs=[
          pl.BlockSpec((BLK_E, D), lambda i: (i, 0)),
          pl.BlockSpec((BLK_E, D), lambda i: (i, 0)),
          pl.BlockSpec((BLK_E, P), lambda i: (i, 0)),
          pl.BlockSpec((BLK_E, P), lambda i: (i, 0)),
          _full((1, D)), _full((D, D)), _full((1, D)),
      ],
      out_specs=pl.BlockSpec((BLK_E, D), lambda i: (i, 0)),
      out_shape=jax.ShapeDtypeStruct((E, D), _f32),
      compiler_params=pltpu.CompilerParams(
          dimension_semantics=("parallel",)),
  )(ga, gb, pr, pc, w1l, we2, be2)


def _node_mlp(h, msg, wn1a, wn1b, bn1, wn2, bn2, gam, bet):
  t = jax.nn.silu(jnp.dot(h, wn1a[...], preferred_element_type=_f32)
                  + jnp.dot(msg, wn1b[...], preferred_element_type=_f32)
                  + bn1[...])
  hn = jnp.dot(t, wn2[...], preferred_element_type=_f32) + bn2[...] + h
  mu = jnp.mean(hn, axis=-1, keepdims=True)
  var = jnp.mean((hn - mu) ** 2, axis=-1, keepdims=True)
  return (hn - mu) * lax.rsqrt(var + 1e-5) * gam[...] + bet[...]


def _node_body_mid(h, pm, pe, posp, wn1a, wn1b, bn1, wn2, bn2, gam, bet,
                   wa_n, wb_n, be1_n, h_o, a_o, b_o, posp_o):
  msg = pm[...]
  ho = _node_mlp(h[...], msg, wn1a, wn1b, bn1, wn2, bn2, gam, bet)
  h_o[...] = ho
  a_o[...] = jnp.dot(ho, wa_n[...], preferred_element_type=_f32) + be1_n[...]
  b_o[...] = jnp.dot(ho, wb_n[...], preferred_element_type=_f32)
  es = pe[...]
  cnt = jnp.maximum(es[:, 2:3], 1.0)
  colid = lax.broadcasted_iota(jnp.int32, (BLK_N, P), 1)
  posp_o[...] = posp[...] + jnp.where(colid < 2, es[:, 0:P] / cnt, 0.0)


def _tc_node_mid(h, pm, pe, posp, wn1a, wn1b, bn1, wn2, bn2, gam, bet,
                 wa_n, wb_n, be1_n):
  return pl.pallas_call(
      _node_body_mid,
      grid=(N // BLK_N,),
      in_specs=[
          pl.BlockSpec((BLK_N, D), lambda i: (i, 0)),
          pl.BlockSpec((BLK_N, D), lambda i: (i, 0)),
          pl.BlockSpec((BLK_N, D), lambda i: (i, 0)),
          pl.BlockSpec((BLK_N, P), lambda i: (i, 0)),
          _full((D, D)), _full((D, D)), _full((1, D)),
          _full((D, D)), _full((1, D)), _full((1, D)), _full((1, D)),
          _full((D, D)), _full((D, D)), _full((1, D)),
      ],
      out_specs=[
          pl.BlockSpec((BLK_N, D), lambda i: (i, 0)),
          pl.BlockSpec((BLK_N, D), lambda i: (i, 0)),
          pl.BlockSpec((BLK_N, D), lambda i: (i, 0)),
          pl.BlockSpec((BLK_N, P), lambda i: (i, 0)),
      ],
      out_shape=[
          jax.ShapeDtypeStruct((N, D), _f32),
          jax.ShapeDtypeStruct((N, D), _f32),
          jax.ShapeDtypeStruct((N, D), _f32),
          jax.ShapeDtypeStruct((N, P), _f32),
      ],
      compiler_params=pltpu.CompilerParams(
          dimension_semantics=("parallel",)),
  )(h, pm, pe, posp, wn1a, wn1b, bn1, wn2, bn2, gam, bet, wa_n, wb_n, be1_n)


def _node_body_last(h, pm, wn1a, wn1b, bn1, wn2, bn2, gam, bet, h_o):
  msg = pm[0] + pm[1]
  h_o[...] = _node_mlp(h[...], msg, wn1a, wn1b, bn1, wn2, bn2, gam, bet)


def _tc_node_last(h, pm, wn1a, wn1b, bn1, wn2, bn2, gam, bet):
  return pl.pallas_call(
      _node_body_last,
      grid=(N // BLK_N,),
      in_specs=[
          pl.BlockSpec((BLK_N, D), lambda i: (i, 0)),
          pl.BlockSpec((NC, BLK_N, D), lambda i: (0, i, 0)),
          _full((D, D)), _full((D, D)), _full((1, D)),
          _full((D, D)), _full((1, D)), _full((1, D)), _full((1, D)),
      ],
      out_specs=pl.BlockSpec((BLK_N, D), lambda i: (i, 0)),
      out_shape=jax.ShapeDtypeStruct((N, D), _f32),
      compiler_params=pltpu.CompilerParams(
          dimension_semantics=("parallel",)),
  )(h, pm, wn1a, wn1b, bn1, wn2, bn2, gam, bet)


def kernel(x, pos, edge_index, params):
  row = edge_index[0]
  col = edge_index[1]
  posp = jnp.pad(pos, ((0, 0), (0, P - 2)))
  z_m = jnp.zeros((ZR, D), _f32)
  z_e = jnp.zeros((ZR, P), _f32)

  def r1(v):
    return v.reshape(1, D)

  layers = params['layers']
  lp0 = layers[0]
  h, a_t, b_t = _tc_init(
      x, params['proj']['W'], r1(params['proj']['b']),
      lp0['We1'][0:D], lp0['We1'][D:2 * D], r1(lp0['be1']))

  for i, lp in enumerate(layers):
    w1l = lp['We1'][2 * D:2 * D + 1]
    if i < 2:
      ga, gb, pr8, pc8 = _sc_gather(row, col, a_t, b_t, posp)
      wc2r = lp['Wc2'].reshape(1, D)
      bc2v = jnp.broadcast_to(lp['bc2'].reshape(1, 1), (1, D))
      m2, ext = _tc_edge_coord(ga, gb, pr8, pc8, w1l, lp['We2'],
                               r1(lp['be2']), lp['Wc1'], r1(lp['bc1']),
                               wc2r, bc2v)
      pm, pe = _sc_scatter_dual(row, z_m, m2, ext)
      nxt = layers[i + 1]
      h, a_t, b_t, posp = _tc_node_mid(
          h, pm, pe, posp,
          lp['Wn1'][0:D], lp['Wn1'][D:2 * D], r1(lp['bn1']),
          lp['Wn2'], r1(lp['bn2']), r1(lp['gamma']), r1(lp['beta']),
          nxt['We1'][0:D], nxt['We1'][D:2 * D], r1(nxt['be1']))
    else:
      ga, gb, pr8, pc8 = _sc_gather(row, col, a_t, b_t, posp)
      m2 = _tc_edge_plain(ga, gb, pr8, pc8, w1l, lp['We2'], r1(lp['be2']))
      pm, _ = _sc_scatter(row, z_m, z_e, m2, None)
      h = _tc_node_last(
          h, pm,
          lp['Wn1'][0:D], lp['Wn1'][D:2 * D], r1(lp['bn1']),
          lp['Wn2'], r1(lp['bn2']), r1(lp['gamma']), r1(lp['beta']))
  return h


# speedup vs baseline: 4.1728x; 1.0004x over previous
import functools

import jax
import jax.numpy as jnp
from jax import lax
from jax.experimental import pallas as pl
from jax.experimental.pallas import tpu as pltpu
from jax.experimental.pallas import tpu_sc as plsc

N = 10000
E = 320000
D = 128
P = 8

NC = 2
NS = 16
NW = NC * NS
EPW = E // NW
CH = 80
NCHUNK = EPW // CH
ZR = 400
NZC = N // ZR

@functools.cache
def _sc_mesh():
  return plsc.VectorSubcoreMesh(
      core_axis_name="c", subcore_axis_name="s", num_cores=NC, num_subcores=NS)


_f32 = jnp.float32


def _sw_pipe(nch, fa, wa, fb, wb):
  fa(0, 0)
  fa(1, 1)
  wa(0)
  fb(0, 0)

  def pair(j, _):
    i = 2 * j
    wb(0)
    fa(i + 2, 0)
    wa(1)
    fb(i + 1, 1)
    wb(1)
    fa(i + 3, 1)
    wa(0)
    fb(i + 2, 0)
    return 0

  lax.fori_loop(0, (nch - 2) // 2, pair, 0)
  if nch % 2:
    wb(0)
    fa(nch - 1, 0)
    wa(1)
    fb(nch - 2, 1)
    wa(0)
    fb(nch - 1, 0)
    wb(1)
    wb(0)
  else:
    wa(1)
    fb(nch - 1, 1)
    wb(0)
    wb(1)


def _sc_gather(row, col, a_t, b_t, posq):
  @functools.partial(
      pl.kernel,
      out_type=[jax.ShapeDtypeStruct((E, D), _f32)] * 2
      + [jax.ShapeDtypeStruct((E, P), _f32)] * 2,
      mesh=_sc_mesh(),
      scratch_types=[
          pltpu.VMEM((EPW,), jnp.int32),
          pltpu.VMEM((EPW,), jnp.int32),
          [pltpu.VMEM((CH, D), _f32) for _ in range(4)],
          [pltpu.VMEM((CH, P), _f32) for _ in range(4)],
          [pltpu.SemaphoreType.DMA for _ in range(2)],
          [pltpu.SemaphoreType.DMA for _ in range(2)],
      ],
      compiler_params=pltpu.CompilerParams(use_tc_tiling_on_sc=False),
  )
  def k(row_h, col_h, a_h, b_h, pq_h, ga_h, gb_h, pr_h, pc_h,
        idx_r, idx_c, bufs, pbufs, sg, sw):
    wid = lax.axis_index("s") * NC + lax.axis_index("c")
    ebase = wid * EPW
    pltpu.sync_copy(row_h.at[pl.ds(ebase, EPW)], idx_r)
    pltpu.sync_copy(col_h.at[pl.ds(ebase, EPW)], idx_c)
    outs = (ga_h, gb_h)
    pouts = (pr_h, pc_h)

    def fire(i, s):
      o = i * CH
      pltpu.async_copy(a_h.at[idx_r.at[pl.ds(o, CH)]], bufs[2 * s], sg[s])
      pltpu.async_copy(b_h.at[idx_c.at[pl.ds(o, CH)]], bufs[2 * s + 1], sg[s])
      pltpu.async_copy(pq_h.at[idx_r.at[pl.ds(o, CH)]], pbufs[2 * s], sg[s])
      pltpu.async_copy(pq_h.at[idx_c.at[pl.ds(o, CH)]], pbufs[2 * s + 1],
                       sg[s])

    def wait_g(s):
      for t in range(2):
        pltpu.make_async_copy(outs[t].at[pl.ds(0, CH)], bufs[2 * s + t],
                              sg[s]).wait()
        pltpu.make_async_copy(pouts[t].at[pl.ds(0, CH)], pbufs[2 * s + t],
                              sg[s]).wait()

    def writeback(i, s):
      base = ebase + i * CH
      for t in range(2):
        pltpu.async_copy(bufs[2 * s + t], outs[t].at[pl.ds(base, CH)], sw[s])
        pltpu.async_copy(pbufs[2 * s + t], pouts[t].at[pl.ds(base, CH)],
                         sw[s])

    def wait_w(s):
      for t in range(2):
        pltpu.make_async_copy(bufs[2 * s + t], outs[t].at[pl.ds(0, CH)],
                              sw[s]).wait()
        pltpu.make_async_copy(pbufs[2 * s + t], pouts[t].at[pl.ds(0, CH)],
                              sw[s]).wait()

    _sw_pipe(NCHUNK, fire, wait_g, writeback, wait_w)

  return k(row, col, a_t, b_t, posq)


_SCAT_SCRATCH = [
    [pltpu.VMEM((CH,), jnp.int32) for _ in range(2)],
    [pltpu.VMEM((CH, D), _f32) for _ in range(2)],
    [pltpu.SemaphoreType.DMA for _ in range(2)],
    [pltpu.SemaphoreType.DMA for _ in range(2)],
    pltpu.VMEM_SHARED((N, D), _f32),
]


def _sc_scatter_dual(row, z_m, m2, ext):
  @functools.partial(
      pl.kernel,
      out_type=[
          jax.ShapeDtypeStruct((N, D), _f32),
          jax.ShapeDtypeStruct((N, D), _f32),
      ],
      mesh=_sc_mesh(),
      scratch_types=_SCAT_SCRATCH,
  )
  def k(row_h, zm_h, m2_h, ext_h, pm_h, pe_h, ix, vb, sl, ss, acc_m):
    cid = lax.axis_index("c")
    sid = lax.axis_index("s")

    def zbody(j, _):
      c = sid + NS * j

      @pl.when(c < NZC)
      def _():
        pltpu.sync_copy(zm_h, acc_m.at[pl.ds(c * ZR, ZR)])
      return 0

    lax.fori_loop(0, (NZC + NS - 1) // NS, zbody, 0)
    plsc.subcore_barrier()
    ebase = sid * EPS

    def mk(val_h):
      def fire_load(i, s):
        base = ebase + i * CH
        pltpu.async_copy(row_h.at[pl.ds(base, CH)], ix[s], sl[s])
        pltpu.async_copy(val_h.at[pl.ds(base, CH)], vb[s], sl[s])

      def wait_load(s):
        pltpu.make_async_copy(row_h.at[pl.ds(0, CH)], ix[s], sl[s]).wait()
        pltpu.make_async_copy(val_h.at[pl.ds(0, CH)], vb[s], sl[s]).wait()

      def fire_scat(i, s):
        pltpu.async_copy(vb[s], acc_m.at[ix[s]], ss[s], add=True)

      def wait_scat(s):
        pltpu.make_async_copy(vb[s], acc_m.at[pl.ds(0, CH)], ss[s]).wait()

      return fire_load, wait_load, fire_scat, wait_scat

    @pl.when(cid == 0)
    def _():
      _sw_pipe(NCHS, *mk(m2_h))

    @pl.when(cid == 1)
    def _():
      _sw_pipe(NCHS, *mk(ext_h))

    plsc.subcore_barrier()

    def wbody(j, _):
      c = sid + NS * j

      @pl.when(c < NZC)
      def _():
        @pl.when(cid == 0)
        def _():
          pltpu.sync_copy(acc_m.at[pl.ds(c * ZR, ZR)],
                          pm_h.at[pl.ds(c * ZR, ZR)])

        @pl.when(cid == 1)
        def _():
          pltpu.sync_copy(acc_m.at[pl.ds(c * ZR, ZR)],
                          pe_h.at[pl.ds(c * ZR, ZR)])
      return 0

    lax.fori_loop(0, (NZC + NS - 1) // NS, wbody, 0)

  return k(row, z_m, m2, ext)


EPS = E // NS
NCHS = EPS // CH


def _sc_scatter(row, z_m, z_e, m2, ext):
  with_ext = ext is not None
  scratch = list(_SCAT_SCRATCH)
  if with_ext:
    scratch += [
        [pltpu.VMEM((CH, P), _f32) for _ in range(2)],
        pltpu.VMEM_SHARED((N, P), _f32),
    ]
  out_type = [jax.ShapeDtypeStruct((NC, N, D), _f32)]
  if with_ext:
    out_type.append(jax.ShapeDtypeStruct((NC, N, P), _f32))

  @functools.partial(
      pl.kernel,
      out_type=out_type,
      mesh=_sc_mesh(),
      scratch_types=scratch,
      compiler_params=pltpu.CompilerParams(use_tc_tiling_on_sc=False),
  )
  def k(*args):
    if with_ext:
      (row_h, zm_h, ze_h, m2_h, ext_h, pm_h, pe_h,
       ix, vb, sl, ss, acc_m, eb, acc_e) = args
    else:
      row_h, zm_h, ze_h, m2_h, pm_h, ix, vb, sl, ss, acc_m = args
    cid = lax.axis_index("c")
    sid = lax.axis_index("s")
    ebase = (sid * NC + cid) * EPW

    def zbody(j, _):
      c = sid + NS * j

      @pl.when(c < NZC)
      def _():
        pltpu.sync_copy(zm_h, acc_m.at[pl.ds(c * ZR, ZR)])
        if with_ext:
          pltpu.sync_copy(ze_h, acc_e.at[pl.ds(c * ZR, ZR)])
      return 0

    lax.fori_loop(0, (NZC + NS - 1) // NS, zbody, 0)
    plsc.subcore_barrier()

    def fire_load(i, s):
      base = ebase + i * CH
      pltpu.async_copy(row_h.at[pl.ds(base, CH)], ix[s], sl[s])
      pltpu.async_copy(m2_h.at[pl.ds(base, CH)], vb[s], sl[s])
      if with_ext:
        pltpu.async_copy(ext_h.at[pl.ds(base, CH)], eb[s], sl[s])

    def wait_load(s):
      pltpu.make_async_copy(row_h.at[pl.ds(0, CH)], ix[s], sl[s]).wait()
      pltpu.make_async_copy(m2_h.at[pl.ds(0, CH)], vb[s], sl[s]).wait()
      if with_ext:
        pltpu.make_async_copy(ext_h.at[pl.ds(0, CH)], eb[s], sl[s]).wait()

    def fire_scat(i, s):
      pltpu.async_copy(vb[s], acc_m.at[ix[s]], ss[s], add=True)
      if with_ext:
        pltpu.async_copy(eb[s], acc_e.at[ix[s]], ss[s], add=True)

    def wait_scat(s):
      pltpu.make_async_copy(vb[s], acc_m.at[pl.ds(0, CH)], ss[s]).wait()
      if with_ext:
        pltpu.make_async_copy(eb[s], acc_e.at[pl.ds(0, CH)], ss[s]).wait()

    _sw_pipe(NCHUNK, fire_load, wait_load, fire_scat, wait_scat)
    plsc.subcore_barrier()

    def wbody(j, _):
      c = sid + NS * j

      @pl.when(c < NZC)
      def _():
        pltpu.sync_copy(acc_m.at[pl.ds(c * ZR, ZR)],
                        pm_h.at[cid, pl.ds(c * ZR, ZR)])
        if with_ext:
          pltpu.sync_copy(acc_e.at[pl.ds(c * ZR, ZR)],
                          pe_h.at[cid, pl.ds(c * ZR, ZR)])
      return 0

    lax.fori_loop(0, (NZC + NS - 1) // NS, wbody, 0)

  if with_ext:
    return k(row, z_m, z_e, m2, ext)
  res = k(row, z_m, z_e, m2)
  if isinstance(res, (list, tuple)):
    res = res[0]
  return res, None


BLK_E = 4000
BLK_N = 1000


def _full(shape):
  return pl.BlockSpec(shape, lambda i: tuple(0 for _ in shape))


def _init_body(x, wp, bp, wa, wb, be1, h_o, a_o, b_o):
  h = jax.nn.silu(jnp.dot(x[...], wp[...], preferred_element_type=_f32)
                  + bp[...])
  h_o[...] = h
  a_o[...] = jnp.dot(h, wa[...], preferred_element_type=_f32) + be1[...]
  b_o[...] = jnp.dot(h, wb[...], preferred_element_type=_f32)


def _tc_init(x, wp, bp, wa, wb, be1):
  return pl.pallas_call(
      _init_body,
      grid=(N // BLK_N,),
      in_specs=[
          pl.BlockSpec((BLK_N, D), lambda i: (i, 0)),
          _full((D, D)), _full((1, D)), _full((D, D)), _full((D, D)),
          _full((1, D)),
      ],
      out_specs=[
          pl.BlockSpec((BLK_N, D), lambda i: (i, 0)),
          pl.BlockSpec((BLK_N, D), lambda i: (i, 0)),
          pl.BlockSpec((BLK_N, D), lambda i: (i, 0)),
      ],
      out_shape=[jax.ShapeDtypeStruct((N, D), _f32)] * 3,
      compiler_params=pltpu.CompilerParams(
          dimension_semantics=("parallel",)),
  )(x, wp, bp, wa, wb, be1)


def _edge_body_coord(ga, gb, pr, pc, w1l, we2, be2, wc1, bc1, wc2r, bc2v,
                     m2_o, ext_o):
  dif = pr[...] - pc[...]
  dist = jnp.sum(dif * dif, axis=-1, keepdims=True)
  m1 = jax.nn.silu(ga[...] + gb[...] + dist * w1l[...])
  m2 = jax.nn.silu(jnp.dot(m1, we2[...], preferred_element_type=_f32)
                   + be2[...])
  m2_o[...] = m2
  c1 = jax.nn.silu(jnp.dot(m2, wc1[...], preferred_element_type=_f32)
                   + bc1[...])
  cw = jnp.sum(c1 * wc2r[...], axis=-1, keepdims=True) + bc2v[...][:, 0:1]
  dx = dif[:, 0:1]
  dy = dif[:, 1:2]
  colid = lax.broadcasted_iota(jnp.int32, (BLK_E, D), 1)
  ext_o[...] = jnp.where(
      colid == 0, dx * cw,
      jnp.where(colid == 1, dy * cw,
                jnp.where(colid == 2, 1.0, 0.0)))


def _tc_edge_coord(ga, gb, pr, pc, w1l, we2, be2, wc1, bc1, wc2r, bc2v):
  return pl.pallas_call(
      _edge_body_coord,
      grid=(E // BLK_E,),
      in_specs=[
          pl.BlockSpec((BLK_E, D), lambda i: (i, 0)),
          pl.BlockSpec((BLK_E, D), lambda i: (i, 0)),
          pl.BlockSpec((BLK_E, P), lambda i: (i, 0)),
          pl.BlockSpec((BLK_E, P), lambda i: (i, 0)),
          _full((1, D)), _full((D, D)), _full((1, D)),
          _full((D, D)), _full((1, D)), _full((1, D)), _full((1, D)),
      ],
      out_specs=[
          pl.BlockSpec((BLK_E, D), lambda i: (i, 0)),
          pl.BlockSpec((BLK_E, D), lambda i: (i, 0)),
      ],
      out_shape=[
          jax.ShapeDtypeStruct((E, D), _f32),
          jax.ShapeDtypeStruct((E, D), _f32),
      ],
      compiler_params=pltpu.CompilerParams(
          dimension_semantics=("parallel",)),
  )(ga, gb, pr, pc, w1l, we2, be2, wc1, bc1, wc2r, bc2v)


def _edge_body_plain(ga, gb, pr, pc, w1l, we2, be2, m2_o):
  dif = pr[...] - pc[...]
  dist = jnp.sum(dif * dif, axis=-1, keepdims=True)
  m1 = jax.nn.silu(ga[...] + gb[...] + dist * w1l[...])
  m2_o[...] = jax.nn.silu(jnp.dot(m1, we2[...], preferred_element_type=_f32)
                          + be2[...])


def _tc_edge_plain(ga, gb, pr, pc, w1l, we2, be2):
  return pl.pallas_call(
      _edge_body_plain,
      grid=(E // BLK_E,),
      in_specs=[
          pl.BlockSpec((BLK_E, D), lambda i: (i, 0)),
          pl.BlockSpec((BLK_E, D), lambda i: (i, 0)),
          pl.BlockSpec((BLK_E, P), lambda i: (i, 0)),
          pl.BlockSpec((BLK_E, P), lambda i: (i, 0)),
          _full((1, D)), _full((D, D)), _full((1, D)),
      ],
      out_specs=pl.BlockSpec((BLK_E, D), lambda i: (i, 0)),
      out_shape=jax.ShapeDtypeStruct((E, D), _f32),
      compiler_params=pltpu.CompilerParams(
          dimension_semantics=("parallel",)),
  )(ga, gb, pr, pc, w1l, we2, be2)


def _node_mlp(h, msg, wn1a, wn1b, bn1, wn2, bn2, gam, bet):
  t = jax.nn.silu(jnp.dot(h, wn1a[...], preferred_element_type=_f32)
                  + jnp.dot(msg, wn1b[...], preferred_element_type=_f32)
                  + bn1[...])
  hn = jnp.dot(t, wn2[...], preferred_element_type=_f32) + bn2[...] + h
  mu = jnp.mean(hn, axis=-1, keepdims=True)
  var = jnp.mean((hn - mu) ** 2, axis=-1, keepdims=True)
  return (hn - mu) * lax.rsqrt(var + 1e-5) * gam[...] + bet[...]


def _node_body_mid(h, pm, pe, posp, wn1a, wn1b, bn1, wn2, bn2, gam, bet,
                   wa_n, wb_n, be1_n, h_o, a_o, b_o, posp_o):
  msg = pm[...]
  ho = _node_mlp(h[...], msg, wn1a, wn1b, bn1, wn2, bn2, gam, bet)
  h_o[...] = ho
  a_o[...] = jnp.dot(ho, wa_n[...], preferred_element_type=_f32) + be1_n[...]
  b_o[...] = jnp.dot(ho, wb_n[...], preferred_element_type=_f32)
  es = pe[...]
  cnt = jnp.maximum(es[:, 2:3], 1.0)
  colid = lax.broadcasted_iota(jnp.int32, (BLK_N, P), 1)
  posp_o[...] = posp[...] + jnp.where(colid < 2, es[:, 0:P] / cnt, 0.0)


def _tc_node_mid(h, pm, pe, posp, wn1a, wn1b, bn1, wn2, bn2, gam, bet,
                 wa_n, wb_n, be1_n):
  return pl.pallas_call(
      _node_body_mid,
      grid=(N // BLK_N,),
      in_specs=[
          pl.BlockSpec((BLK_N, D), lambda i: (i, 0)),
          pl.BlockSpec((BLK_N, D), lambda i: (i, 0)),
          pl.BlockSpec((BLK_N, D), lambda i: (i, 0)),
          pl.BlockSpec((BLK_N, P), lambda i: (i, 0)),
          _full((D, D)), _full((D, D)), _full((1, D)),
          _full((D, D)), _full((1, D)), _full((1, D)), _full((1, D)),
          _full((D, D)), _full((D, D)), _full((1, D)),
      ],
      out_specs=[
          pl.BlockSpec((BLK_N, D), lambda i: (i, 0)),
          pl.BlockSpec((BLK_N, D), lambda i: (i, 0)),
          pl.BlockSpec((BLK_N, D), lambda i: (i, 0)),
          pl.BlockSpec((BLK_N, P), lambda i: (i, 0)),
      ],
      out_shape=[
          jax.ShapeDtypeStruct((N, D), _f32),
          jax.ShapeDtypeStruct((N, D), _f32),
          jax.ShapeDtypeStruct((N, D), _f32),
          jax.ShapeDtypeStruct((N, P), _f32),
      ],
      compiler_params=pltpu.CompilerParams(
          dimension_semantics=("parallel",)),
  )(h, pm, pe, posp, wn1a, wn1b, bn1, wn2, bn2, gam, bet, wa_n, wb_n, be1_n)


def _node_body_last(h, pm, wn1a, wn1b, bn1, wn2, bn2, gam, bet, h_o):
  msg = pm[0] + pm[1]
  h_o[...] = _node_mlp(h[...], msg, wn1a, wn1b, bn1, wn2, bn2, gam, bet)


def _tc_node_last(h, pm, wn1a, wn1b, bn1, wn2, bn2, gam, bet):
  return pl.pallas_call(
      _node_body_last,
      grid=(N // BLK_N,),
      in_specs=[
          pl.BlockSpec((BLK_N, D), lambda i: (i, 0)),
          pl.BlockSpec((NC, BLK_N, D), lambda i: (0, i, 0)),
          _full((D, D)), _full((D, D)), _full((1, D)),
          _full((D, D)), _full((1, D)), _full((1, D)), _full((1, D)),
      ],
      out_specs=pl.BlockSpec((BLK_N, D), lambda i: (i, 0)),
      out_shape=jax.ShapeDtypeStruct((N, D), _f32),
      compiler_params=pltpu.CompilerParams(
          dimension_semantics=("parallel",)),
  )(h, pm, wn1a, wn1b, bn1, wn2, bn2, gam, bet)


def kernel(x, pos, edge_index, params):
  row = edge_index[0]
  col = edge_index[1]
  posp = jnp.pad(pos, ((0, 0), (0, P - 2)))
  z_m = jnp.zeros((ZR, D), _f32)
  z_e = jnp.zeros((ZR, P), _f32)

  def r1(v):
    return v.reshape(1, D)

  layers = params['layers']
  lp0 = layers[0]
  h, a_t, b_t = _tc_init(
      x, params['proj']['W'], r1(params['proj']['b']),
      lp0['We1'][0:D], lp0['We1'][D:2 * D], r1(lp0['be1']))

  for i, lp in enumerate(layers):
    w1l = lp['We1'][2 * D:2 * D + 1]
    if i < 2:
      ga, gb, pr8, pc8 = _sc_gather(row, col, a_t, b_t, posp)
      wc2r = lp['Wc2'].reshape(1, D)
      bc2v = jnp.broadcast_to(lp['bc2'].reshape(1, 1), (1, D))
      m2, ext = _tc_edge_coord(ga, gb, pr8, pc8, w1l, lp['We2'],
                               r1(lp['be2']), lp['Wc1'], r1(lp['bc1']),
                               wc2r, bc2v)
      pm, pe = _sc_scatter_dual(row, z_m, m2, ext)
      nxt = layers[i + 1]
      h, a_t, b_t, posp = _tc_node_mid(
          h, pm, pe, posp,
          lp['Wn1'][0:D], lp['Wn1'][D:2 * D], r1(lp['bn1']),
          lp['Wn2'], r1(lp['bn2']), r1(lp['gamma']), r1(lp['beta']),
          nxt['We1'][0:D], nxt['We1'][D:2 * D], r1(nxt['be1']))
    else:
      ga, gb, pr8, pc8 = _sc_gather(row, col, a_t, b_t, posp)
      m2 = _tc_edge_plain(ga, gb, pr8, pc8, w1l, lp['We2'], r1(lp['be2']))
      pm, _ = _sc_scatter(row, z_m, z_e, m2, None)
      h = _tc_node_last(
          h, pm,
          lp['Wn1'][0:D], lp['Wn1'][D:2 * D], r1(lp['bn1']),
          lp['Wn2'], r1(lp['bn2']), r1(lp['gamma']), r1(lp['beta']))
  return h
